# Initial kernel scaffold; baseline (speedup 1.0000x reference)
import jax
import jax.numpy as jnp
from jax.experimental import pallas as pl


def kernel(x, edge_index, batch, W1, b1, W_mu, b_mu, W_lv, b_lv):
    raise NotImplementedError("write your pallas kernel here")


# speedup vs baseline: 14.7035x; 14.7035x over previous
import functools

import jax
import jax.numpy as jnp
from jax import lax
from jax.experimental import pallas as pl
from jax.experimental.pallas import tpu as pltpu
from jax.experimental.pallas import tpu_sc as plsc

NC = 2
NS = 16
L = 16
NW = NC * NS


def _pick_batch(ept):
    for b in range(128, 7, -8):
        if ept % b == 0:
            return b
    raise ValueError(f"no batch size divides edges-per-tile {ept}")


def _zero_rows(ref, nrows, ncols):
    zz = jnp.zeros((L,), jnp.float32)

    def body(r, _):
        for cc in range(ncols // L):
            ref[r, pl.ds(cc * L, L)] = zz
        return _

    lax.fori_loop(0, nrows, body, None)


def _deg_kernel(n, d, ept, B):
    nb = ept // B
    rpt = n // NS
    CH = 64
    nfull, tail = rpt // CH, rpt % CH
    mesh = plsc.VectorSubcoreMesh(core_axis_name="c", subcore_axis_name="s")

    @functools.partial(
        pl.kernel,
        out_type=jax.ShapeDtypeStruct((NC, n, d), jnp.float32),
        mesh=mesh,
        scratch_types=[
            pltpu.VMEM((B,), jnp.int32),
            pltpu.VMEM((B, d), jnp.float32),
            pltpu.VMEM((CH, d), jnp.float32),
            pltpu.VMEM_SHARED((n, d), jnp.float32),
        ],
    )
    def k(dst_hbm, out_hbm, idx_v, ones_v, zbuf, hist):
        cid = lax.axis_index("c")
        sid = lax.axis_index("s")
        wid = cid * NS + sid
        base = sid * rpt

        _zero_rows(zbuf, CH, d)
        one = jnp.ones((L,), jnp.float32)

        def fill(i, _):
            for cc in range(d // L):
                ones_v[i, pl.ds(cc * L, L)] = one
            return _

        lax.fori_loop(0, B, fill, None)
        for t in range(nfull):
            pltpu.sync_copy(zbuf, hist.at[pl.ds(base + t * CH, CH)])
        if tail:
            pltpu.sync_copy(zbuf.at[pl.ds(0, tail)],
                            hist.at[pl.ds(base + nfull * CH, tail)])
        plsc.subcore_barrier()

        ebase = wid * ept

        def step(j, _):
            pltpu.sync_copy(dst_hbm.at[pl.ds(ebase + j * B, B)], idx_v)
            pltpu.sync_copy(ones_v, hist.at[idx_v], add=True)
            return _

        lax.fori_loop(0, nb, step, None)
        plsc.subcore_barrier()
        for t in range(nfull):
            pltpu.sync_copy(hist.at[pl.ds(base + t * CH, CH)], zbuf)
            pltpu.sync_copy(zbuf, out_hbm.at[cid, pl.ds(base + t * CH, CH)])
        if tail:
            pltpu.sync_copy(hist.at[pl.ds(base + nfull * CH, tail)],
                            zbuf.at[pl.ds(0, tail)])
            pltpu.sync_copy(zbuf.at[pl.ds(0, tail)],
                            out_hbm.at[cid, pl.ds(base + nfull * CH, tail)])

    return k


def _scatter_kernel(n, d, ept, B):
    nb = ept // B
    rpt = n // NS
    CH = 64
    nfull, tail = rpt // CH, rpt % CH
    mesh = plsc.VectorSubcoreMesh(core_axis_name="c", subcore_axis_name="s")

    @functools.partial(
        pl.kernel,
        out_type=jax.ShapeDtypeStruct((NC, n, d), jnp.float32),
        mesh=mesh,
        scratch_types=[
            pltpu.VMEM((B,), jnp.int32),
            pltpu.VMEM((B,), jnp.int32),
            pltpu.VMEM((B, d), jnp.float32),
            pltpu.VMEM((CH, d), jnp.float32),
            pltpu.VMEM_SHARED((n, d), jnp.float32),
            pltpu.SemaphoreType.DMA,
        ],
    )
    def k(xt_hbm, src_hbm, dst_hbm, out_hbm, sidx, didx, rows, zbuf, acc, sem):
        cid = lax.axis_index("c")
        sid = lax.axis_index("s")
        wid = cid * NS + sid
        base = sid * rpt

        _zero_rows(zbuf, CH, d)
        for t in range(nfull):
            pltpu.sync_copy(zbuf, acc.at[pl.ds(base + t * CH, CH)])
        if tail:
            pltpu.sync_copy(zbuf.at[pl.ds(0, tail)],
                            acc.at[pl.ds(base + nfull * CH, tail)])
        plsc.subcore_barrier()

        ebase = wid * ept

        def step(j, _):
            pltpu.sync_copy(src_hbm.at[pl.ds(ebase + j * B, B)], sidx)
            pltpu.sync_copy(dst_hbm.at[pl.ds(ebase + j * B, B)], didx)
            pltpu.async_copy(xt_hbm.at[sidx], rows, sem).wait()
            pltpu.sync_copy(rows, acc.at[didx], add=True)
            return _

        lax.fori_loop(0, nb, step, None)
        plsc.subcore_barrier()

        for t in range(nfull):
            pltpu.sync_copy(acc.at[pl.ds(base + t * CH, CH)], zbuf)
            pltpu.sync_copy(zbuf, out_hbm.at[cid, pl.ds(base + t * CH, CH)])
        if tail:
            pltpu.sync_copy(acc.at[pl.ds(base + nfull * CH, tail)],
                            zbuf.at[pl.ds(0, tail)])
            pltpu.sync_copy(zbuf.at[pl.ds(0, tail)],
                            out_hbm.at[cid, pl.ds(base + nfull * CH, tail)])

    return k


def _scatter_kernel_v2(n, d, ept, B):
    nb = ept // B
    assert nb % 2 == 1
    rpt = n // NS
    CH = 64
    nfull, tail = rpt // CH, rpt % CH
    mesh = plsc.VectorSubcoreMesh(core_axis_name="c", subcore_axis_name="s")

    @functools.partial(
        pl.kernel,
        out_type=jax.ShapeDtypeStruct((NC, n, d), jnp.float32),
        mesh=mesh,
        scratch_types=[
            pltpu.VMEM((B,), jnp.int32),
            pltpu.VMEM((B,), jnp.int32),
            pltpu.VMEM((B,), jnp.int32),
            pltpu.VMEM((B,), jnp.int32),
            pltpu.VMEM((B, d), jnp.float32),
            pltpu.VMEM((B, d), jnp.float32),
            pltpu.VMEM((CH, d), jnp.float32),
            pltpu.VMEM_SHARED((n, d), jnp.float32),
            pltpu.SemaphoreType.DMA,
            pltpu.SemaphoreType.DMA,
            pltpu.SemaphoreType.DMA,
            pltpu.SemaphoreType.DMA,
        ],
    )
    def k(xt_hbm, src_hbm, dst_hbm, out_hbm,
          s0, d0, s1, d1, r0, r1, zbuf, acc, gs0, gs1, ss0, ss1):
        cid = lax.axis_index("c")
        sid = lax.axis_index("s")
        wid = cid * NS + sid
        base = sid * rpt

        _zero_rows(zbuf, CH, d)
        for t in range(nfull):
            pltpu.sync_copy(zbuf, acc.at[pl.ds(base + t * CH, CH)])
        if tail:
            pltpu.sync_copy(zbuf.at[pl.ds(0, tail)],
                            acc.at[pl.ds(base + nfull * CH, tail)])
        plsc.subcore_barrier()

        ebase = wid * ept
        bufs = ((s0, d0, r0, gs0, ss0), (s1, d1, r1, gs1, ss1))

        def start_batch(j, bu):
            s_, d_, r_, gs_, _ = bu
            pltpu.sync_copy(src_hbm.at[pl.ds(ebase + j * B, B)], s_)
            pltpu.sync_copy(dst_hbm.at[pl.ds(ebase + j * B, B)], d_)
            pltpu.make_async_copy(xt_hbm.at[s_], r_, gs_).start()

        def wait_gather(bu):
            s_, _, r_, gs_, _ = bu
            pltpu.make_async_copy(xt_hbm.at[s_], r_, gs_).wait()

        def start_scat(bu):
            _, d_, r_, _, ss_ = bu
            pltpu.make_async_copy(r_, acc.at[d_], ss_).start(add=True)

        def wait_scat(bu):
            _, d_, r_, _, ss_ = bu
            pltpu.make_async_copy(r_, acc.at[d_], ss_).wait()

        start_batch(0, bufs[0])
        start_batch(1, bufs[1])

        def pair(p, _):
            j0 = 2 * p
            wait_gather(bufs[0])
            start_scat(bufs[0])
            wait_gather(bufs[1])
            start_scat(bufs[1])
            wait_scat(bufs[0])
            start_batch(j0 + 2, bufs[0])
            wait_scat(bufs[1])
            start_batch(j0 + 3, bufs[1])
            return _

        nsteady = (nb - 3) // 2
        lax.fori_loop(0, nsteady, pair, None)
        wait_gather(bufs[0])
        start_scat(bufs[0])
        wait_gather(bufs[1])
        start_scat(bufs[1])
        wait_scat(bufs[0])
        start_batch(nb - 1, bufs[0])
        wait_scat(bufs[1])
        wait_gather(bufs[0])
        start_scat(bufs[0])
        wait_scat(bufs[0])
        plsc.subcore_barrier()

        for t in range(nfull):
            pltpu.sync_copy(acc.at[pl.ds(base + t * CH, CH)], zbuf)
            pltpu.sync_copy(zbuf, out_hbm.at[cid, pl.ds(base + t * CH, CH)])
        if tail:
            pltpu.sync_copy(acc.at[pl.ds(base + nfull * CH, tail)],
                            zbuf.at[pl.ds(0, tail)])
            pltpu.sync_copy(zbuf.at[pl.ds(0, tail)],
                            out_hbm.at[cid, pl.ds(base + nfull * CH, tail)])

    return k


def _tc_pre(n, d, rb):
    def body(degp_ref, x_ref, xt_ref, dinv_ref):
        v = degp_ref[...]
        deg = v[0, :, 0:1] + v[1, :, 0:1] + 1.0
        dinv = lax.rsqrt(deg)
        xt_ref[...] = x_ref[...] * dinv
        dinv_ref[...] = dinv

    return pl.pallas_call(
        body,
        grid=(n // rb,),
        in_specs=[
            pl.BlockSpec((NC, rb, d), lambda i: (0, i, 0)),
            pl.BlockSpec((rb, d), lambda i: (i, 0)),
        ],
        out_specs=[
            pl.BlockSpec((rb, d), lambda i: (i, 0)),
            pl.BlockSpec((rb, 1), lambda i: (i, 0)),
        ],
        out_shape=[
            jax.ShapeDtypeStruct((n, d), jnp.float32),
            jax.ShapeDtypeStruct((n, 1), jnp.float32),
        ],
    )


def _tc_mid(n, d, rb):
    def body(dinv_ref, acc_ref, xt_ref, w_ref, b_ref, ht_ref):
        dinv = dinv_ref[...]
        a = acc_ref[...]
        g1 = (a[0] + a[1] + xt_ref[...]) * dinv
        h = jnp.dot(g1, w_ref[...], preferred_element_type=jnp.float32,
                    precision=lax.Precision.HIGHEST)
        h = jnp.maximum(h + b_ref[...], 0.0)
        ht_ref[...] = h * dinv

    return pl.pallas_call(
        body,
        grid=(n // rb,),
        in_specs=[
            pl.BlockSpec((rb, 1), lambda i: (i, 0)),
            pl.BlockSpec((NC, rb, d), lambda i: (0, i, 0)),
            pl.BlockSpec((rb, d), lambda i: (i, 0)),
            pl.BlockSpec((d, d), lambda i: (0, 0)),
            pl.BlockSpec((1, d), lambda i: (0, 0)),
        ],
        out_specs=pl.BlockSpec((rb, d), lambda i: (i, 0)),
        out_shape=jax.ShapeDtypeStruct((n, d), jnp.float32),
    )


def _tc_out(n, d, dl, rb):
    def body(dinv_ref, acc_ref, ht_ref, wm_ref, bm_ref, wl_ref, bl_ref,
             mu_ref, lv_ref):
        dinv = dinv_ref[...]
        a = acc_ref[...]
        g2 = (a[0] + a[1] + ht_ref[...]) * dinv
        mu_ref[...] = jnp.dot(g2, wm_ref[...],
                              preferred_element_type=jnp.float32,
                              precision=lax.Precision.HIGHEST) + bm_ref[...]
        lv_ref[...] = jnp.dot(g2, wl_ref[...],
                              preferred_element_type=jnp.float32,
                              precision=lax.Precision.HIGHEST) + bl_ref[...]

    return pl.pallas_call(
        body,
        grid=(n // rb,),
        in_specs=[
            pl.BlockSpec((rb, 1), lambda i: (i, 0)),
            pl.BlockSpec((NC, rb, d), lambda i: (0, i, 0)),
            pl.BlockSpec((rb, d), lambda i: (i, 0)),
            pl.BlockSpec((d, dl), lambda i: (0, 0)),
            pl.BlockSpec((1, dl), lambda i: (0, 0)),
            pl.BlockSpec((d, dl), lambda i: (0, 0)),
            pl.BlockSpec((1, dl), lambda i: (0, 0)),
        ],
        out_specs=[
            pl.BlockSpec((rb, dl), lambda i: (i, 0)),
            pl.BlockSpec((rb, dl), lambda i: (i, 0)),
        ],
        out_shape=[
            jax.ShapeDtypeStruct((n, dl), jnp.float32),
            jax.ShapeDtypeStruct((n, dl), jnp.float32),
        ],
    )


def kernel(x, edge_index, batch, W1, b1, W_mu, b_mu, W_lv, b_lv):
    n, d = x.shape
    dl = W_mu.shape[1]
    e = edge_index.shape[1]
    assert e % NW == 0
    ept = e // NW
    B = _pick_batch(ept)
    npad = -(-n // (NS * 8)) * (NS * 8)

    src = edge_index[0].astype(jnp.int32)
    dst = edge_index[1].astype(jnp.int32)

    rb = 1000 if n % 1000 == 0 else n
    b1r = b1.reshape(1, d)
    bmr = b_mu.reshape(1, dl)
    blr = b_lv.reshape(1, dl)

    degp = _deg_kernel(npad, d, ept, B)(dst)
    xt, dinv = _tc_pre(n, d, rb)(degp, x)
    acc1 = _scatter_kernel(npad, d, ept, B)(xt, src, dst)
    ht = _tc_mid(n, d, rb)(dinv, acc1, xt, W1, b1r)
    acc2 = _scatter_kernel(npad, d, ept, B)(ht, src, dst)
    mu, lv = _tc_out(n, d, dl, rb)(dinv, acc2, ht, W_mu, bmr, W_lv, blr)
    return (mu, lv)
